# Initial kernel scaffold; baseline (speedup 1.0000x reference)
import jax
import jax.numpy as jnp
from jax.experimental import pallas as pl


def kernel(x, edge_index, edge_weight, edge_weight_multiplier, dm_indices, fc_W, fc_b):
    raise NotImplementedError("write your pallas kernel here")


# speedup vs baseline: 71.1927x; 71.1927x over previous
import dataclasses
import functools

import jax
import jax.numpy as jnp
from jax import lax
from jax.experimental import pallas as pl
from jax.experimental.pallas import tpu as pltpu
from jax.experimental.pallas import tpu_sc as plsc

N = 50000
E = 1600000
D = 512
NCLS = 10
NPASS = 4

LANES = 16
ROWS = 3200
NW = 32
EPW = E // NW
CH = 2000
NCH = EPW // CH

_mesh = plsc.VectorSubcoreMesh(core_axis_name="c", subcore_axis_name="s")

_sc_params = pltpu.CompilerParams()
for _f, _v in (("needs_layout_passes", False), ("use_tc_tiling_on_sc", False)):
    if _f in pltpu.CompilerParams.__dataclass_fields__:
        _sc_params = dataclasses.replace(_sc_params, **{_f: _v})


@functools.partial(
    pl.kernel,
    compiler_params=_sc_params,
    out_type=(
        jax.ShapeDtypeStruct((NW, ROWS, LANES), jnp.float32),
        jax.ShapeDtypeStruct((NW, D), jnp.float32),
    ),
    mesh=_mesh,
    scratch_types=[
        pltpu.VMEM((ROWS, LANES), jnp.float32),
        pltpu.VMEM((ROWS, LANES), jnp.float32),
        pltpu.VMEM((CH,), jnp.int32),
        pltpu.VMEM((CH,), jnp.int32),
        pltpu.VMEM((CH,), jnp.float32),
        pltpu.VMEM((D,), jnp.int32),
        pltpu.VMEM((D,), jnp.float32),
        pltpu.SemaphoreType.DMA,
        pltpu.SemaphoreType.DMA,
    ],
)
def _sc_pass(h_hbm, src_hbm, dst_hbm, w_hbm, dm_hbm, part_hbm, selp_hbm,
             h_old, h_acc, sbuf, dbuf, wbuf, dmbuf, selbuf, sem0, sem1):
    cid = lax.axis_index("c")
    sid = lax.axis_index("s")
    wid = sid * 2 + cid

    cp_h = pltpu.async_copy(h_hbm, h_old, sem0)
    cp_dm = pltpu.async_copy(dm_hbm, dmbuf, sem1)

    @pl.loop(0, ROWS, step=8)
    def _zero(j):
        z = jnp.zeros((LANES,), jnp.float32)
        for u in range(8):
            h_acc[j + u, :] = z

    cp_h.wait()
    cp_dm.wait()

    ebase = wid * EPW

    @pl.loop(0, NCH)
    def _chunk(c):
        off = ebase + c * CH
        pltpu.sync_copy(src_hbm.at[pl.ds(off, CH)], sbuf)
        pltpu.sync_copy(dst_hbm.at[pl.ds(off, CH)], dbuf)
        pltpu.sync_copy(w_hbm.at[pl.ds(off, CH)], wbuf)

        @pl.loop(0, CH, step=LANES)
        def _edges(i):
            s16 = sbuf[pl.ds(i, LANES)]
            d16 = dbuf[pl.ds(i, LANES)]
            w16 = wbuf[pl.ds(i, LANES)]
            g = plsc.load_gather(h_old, [s16 >> 4, s16 & 15])
            plsc.addupdate_scatter(h_acc, [d16 >> 4, d16 & 15], g * w16)

    @pl.loop(0, D, step=LANES)
    def _sel(k):
        i16 = dmbuf[pl.ds(k, LANES)]
        selbuf[pl.ds(k, LANES)] = plsc.load_gather(h_acc, [i16 >> 4, i16 & 15])

    pltpu.sync_copy(h_acc, part_hbm.at[wid])
    pltpu.sync_copy(selbuf, selp_hbm.at[wid])


def _tc_reduce(parts):
    cols = ROWS * LANES // 8

    def body(p_ref, o_ref):
        o_ref[...] = jnp.sum(p_ref[...], axis=0, keepdims=True)

    return pl.pallas_call(
        body,
        out_shape=jax.ShapeDtypeStruct((1, ROWS * LANES), jnp.float32),
        grid=(8,),
        in_specs=[pl.BlockSpec((NW, cols), lambda i: (0, i))],
        out_specs=pl.BlockSpec((1, cols), lambda i: (0, i)),
    )(parts)


def _tc_effw(w, m):
    rows, cols = 3125, 512

    def body(w_ref, m_ref, o_ref):
        o_ref[...] = w_ref[...] * jax.nn.sigmoid(m_ref[...])

    out = pl.pallas_call(
        body,
        out_shape=jax.ShapeDtypeStruct((rows, cols), jnp.float32),
        grid=(4,),
        in_specs=[pl.BlockSpec((rows, cols // 4), lambda i: (0, i))] * 2,
        out_specs=pl.BlockSpec((rows, cols // 4), lambda i: (0, i)),
    )(w.reshape(rows, cols), m.reshape(rows, cols))
    return out.reshape(E)


def _tc_head(selp, fc_W, fc_b):
    def body(sp_ref, w_ref, b_ref, o_ref):
        s = jnp.sum(sp_ref[...], axis=0, keepdims=True)
        nrm = jnp.sqrt(jnp.sum(s * s))
        y = lax.dot_general(s, w_ref[...], (((1,), (1,)), ((), ())),
                            precision=lax.Precision.HIGHEST,
                            preferred_element_type=jnp.float32)
        o_ref[...] = y / nrm + b_ref[...]

    out = pl.pallas_call(
        body,
        out_shape=jax.ShapeDtypeStruct((1, NCLS), jnp.float32),
    )(selp, fc_W, fc_b.reshape(1, NCLS))
    return out.reshape(NCLS)


def kernel(x, edge_index, edge_weight, edge_weight_multiplier, dm_indices,
           fc_W, fc_b):
    src = edge_index[0]
    dst = edge_index[1]
    effw = _tc_effw(edge_weight, edge_weight_multiplier)

    h = (jnp.zeros((ROWS * LANES,), jnp.float32)
         .at[:N].set(x.reshape(-1))
         .reshape(ROWS, LANES))

    selp = None
    for p in range(NPASS):
        parts, selp = _sc_pass(h, src, dst, effw, dm_indices)
        if p < NPASS - 1:
            h = _tc_reduce(parts.reshape(NW, ROWS * LANES)).reshape(ROWS, LANES)

    return _tc_head(selp, fc_W, fc_b)


# speedup vs baseline: 144.6698x; 2.0321x over previous
import dataclasses
import functools

import jax
import jax.numpy as jnp
from jax import lax
from jax.experimental import pallas as pl
from jax.experimental.pallas import tpu as pltpu
from jax.experimental.pallas import tpu_sc as plsc

N = 50000
E = 1600000
D = 512
NCLS = 10
NPASS = 4

LANES = 16
HPAD = 51200
NW = 32
EPW = E // NW
CH = 2000
NCH = EPW // CH

_mesh = plsc.VectorSubcoreMesh(core_axis_name="c", subcore_axis_name="s")

_sc_params = pltpu.CompilerParams()
for _f, _v in (("needs_layout_passes", False), ("use_tc_tiling_on_sc", False)):
    if _f in pltpu.CompilerParams.__dataclass_fields__:
        _sc_params = dataclasses.replace(_sc_params, **{_f: _v})


@functools.partial(
    pl.kernel,
    compiler_params=_sc_params,
    out_type=(
        jax.ShapeDtypeStruct((NW, HPAD), jnp.float32),
        jax.ShapeDtypeStruct((NW, D), jnp.float32),
    ),
    mesh=_mesh,
    scratch_types=[
        pltpu.VMEM((HPAD,), jnp.float32),
        pltpu.VMEM((HPAD,), jnp.float32),
        pltpu.VMEM((CH,), jnp.int32),
        pltpu.VMEM((CH,), jnp.int32),
        pltpu.VMEM((CH,), jnp.float32),
        pltpu.VMEM((CH,), jnp.int32),
        pltpu.VMEM((CH,), jnp.int32),
        pltpu.VMEM((CH,), jnp.float32),
        pltpu.VMEM((D,), jnp.int32),
        pltpu.VMEM((D,), jnp.float32),
        pltpu.SemaphoreType.DMA,
        pltpu.SemaphoreType.DMA,
        pltpu.SemaphoreType.DMA,
    ],
)
def _sc_pass(h_hbm, src_hbm, dst_hbm, w_hbm, dm_hbm, part_hbm, selp_hbm,
             h_old, h_acc, sb0, db0, wb0, sb1, db1, wb1, dmbuf, selbuf,
             sem_h, sem_e0, sem_e1):
    cid = lax.axis_index("c")
    sid = lax.axis_index("s")
    wid = sid * 2 + cid
    ebase = wid * EPW

    def start(c, sb, db, wb, sem):
        off = ebase + c * CH
        pltpu.async_copy(src_hbm.at[pl.ds(off, CH)], sb, sem)
        pltpu.async_copy(dst_hbm.at[pl.ds(off, CH)], db, sem)
        pltpu.async_copy(w_hbm.at[pl.ds(off, CH)], wb, sem)

    def wait(sb, db, wb, sem):
        pltpu.make_async_copy(src_hbm.at[pl.ds(0, CH)], sb, sem).wait()
        pltpu.make_async_copy(dst_hbm.at[pl.ds(0, CH)], db, sem).wait()
        pltpu.make_async_copy(w_hbm.at[pl.ds(0, CH)], wb, sem).wait()

    def one_group(sb, db, wb, o):
        s16 = sb[pl.ds(o, LANES)]
        d16 = db[pl.ds(o, LANES)]
        w16 = wb[pl.ds(o, LANES)]
        g = plsc.load_gather(h_old, [s16])
        plsc.addupdate_scatter(h_acc, [d16], g * w16)

    def compute(sb, db, wb):
        @pl.loop(0, CH - LANES, step=2 * LANES)
        def _edges(i):
            one_group(sb, db, wb, i)
            one_group(sb, db, wb, i + LANES)

        one_group(sb, db, wb, CH - LANES)

    cp_h = pltpu.async_copy(h_hbm, h_old, sem_h)
    cp_dm = pltpu.async_copy(dm_hbm, dmbuf, sem_h)
    start(0, sb0, db0, wb0, sem_e0)
    start(1, sb1, db1, wb1, sem_e1)

    @pl.loop(0, HPAD, step=8 * LANES)
    def _zero(j):
        z = jnp.zeros((LANES,), jnp.float32)
        for u in range(8):
            h_acc[pl.ds(j + u * LANES, LANES)] = z

    cp_h.wait()
    cp_dm.wait()

    @pl.loop(0, NCH - 3, step=2)
    def _chunk(c):
        wait(sb0, db0, wb0, sem_e0)
        compute(sb0, db0, wb0)
        start(c + 2, sb0, db0, wb0, sem_e0)
        wait(sb1, db1, wb1, sem_e1)
        compute(sb1, db1, wb1)
        start(c + 3, sb1, db1, wb1, sem_e1)

    wait(sb0, db0, wb0, sem_e0)
    compute(sb0, db0, wb0)
    start(NCH - 1, sb0, db0, wb0, sem_e0)
    wait(sb1, db1, wb1, sem_e1)
    compute(sb1, db1, wb1)
    wait(sb0, db0, wb0, sem_e0)
    compute(sb0, db0, wb0)

    @pl.loop(0, D, step=LANES)
    def _sel(k):
        i16 = dmbuf[pl.ds(k, LANES)]
        selbuf[pl.ds(k, LANES)] = plsc.load_gather(h_acc, [i16])

    pltpu.sync_copy(h_acc, part_hbm.at[wid])
    pltpu.sync_copy(selbuf, selp_hbm.at[wid])


def _tc_reduce(parts):
    cols = HPAD // 10

    def body(p_ref, o_ref):
        o_ref[...] = jnp.sum(p_ref[...], axis=0)

    return pl.pallas_call(
        body,
        out_shape=jax.ShapeDtypeStruct((HPAD,), jnp.float32),
        grid=(10,),
        in_specs=[pl.BlockSpec((NW, cols), lambda i: (0, i))],
        out_specs=pl.BlockSpec((cols,), lambda i: (i,)),
    )(parts)


def _tc_effw(w, m):
    rows, cols = 3125, 512

    def body(w_ref, m_ref, o_ref):
        o_ref[...] = w_ref[...] * jax.nn.sigmoid(m_ref[...])

    out = pl.pallas_call(
        body,
        out_shape=jax.ShapeDtypeStruct((rows, cols), jnp.float32),
        grid=(4,),
        in_specs=[pl.BlockSpec((rows, cols // 4), lambda i: (0, i))] * 2,
        out_specs=pl.BlockSpec((rows, cols // 4), lambda i: (0, i)),
    )(w.reshape(rows, cols), m.reshape(rows, cols))
    return out.reshape(E)


def _tc_head(selp, fc_W, fc_b):
    def body(sp_ref, w_ref, b_ref, o_ref):
        s = jnp.sum(sp_ref[...], axis=0, keepdims=True)
        nrm = jnp.sqrt(jnp.sum(s * s))
        y = lax.dot_general(s, w_ref[...], (((1,), (1,)), ((), ())),
                            precision=lax.Precision.HIGHEST,
                            preferred_element_type=jnp.float32)
        o_ref[...] = y / nrm + b_ref[...]

    out = pl.pallas_call(
        body,
        out_shape=jax.ShapeDtypeStruct((1, NCLS), jnp.float32),
    )(selp, fc_W, fc_b.reshape(1, NCLS))
    return out.reshape(NCLS)


def kernel(x, edge_index, edge_weight, edge_weight_multiplier, dm_indices,
           fc_W, fc_b):
    src = edge_index[0]
    dst = edge_index[1]
    effw = _tc_effw(edge_weight, edge_weight_multiplier)

    h = jnp.zeros((HPAD,), jnp.float32).at[:N].set(x.reshape(-1))

    selp = None
    for p in range(NPASS):
        parts, selp = _sc_pass(h, src, dst, effw, dm_indices)
        if p < NPASS - 1:
            h = _tc_reduce(parts)

    return _tc_head(selp, fc_W, fc_b)
